# Initial kernel scaffold; baseline (speedup 1.0000x reference)
import jax
import jax.numpy as jnp
from jax.experimental import pallas as pl


def kernel(x, a, kernel, bias, Wd, bd):
    raise NotImplementedError("write your pallas kernel here")


# speedup vs baseline: 1.8015x; 1.8015x over previous
import functools

import jax
import jax.numpy as jnp
from jax.experimental import pallas as pl
from jax.experimental.pallas import tpu as pltpu

N = 10000
F = 128
H = 16
N_LABELS = 10

BR = 200
NR = N // BR


def _prep_kernel(x_ref, w_ref, b_ref, yaug_ref, h0_ref):
    x = x_ref[...]
    k1 = w_ref[:F, :]
    k2 = w_ref[F:, :]
    y = jnp.dot(x, k2, preferred_element_type=jnp.float32)
    yaug_ref[:, :H] = y
    yaug_ref[:, H:H + 1] = jnp.ones((N, 1), dtype=jnp.float32)
    yaug_ref[:, H + 1:] = jnp.zeros((N, 32 - H - 1), dtype=jnp.float32)
    h0_ref[...] = jnp.dot(x, k1, preferred_element_type=jnp.float32) + b_ref[...]


def _main_kernel(a_ref, yaug_ref, h0_ref, wd_ref, bd_ref, out_ref,
                 pooled_ref):
    i = pl.program_id(0)

    zaug = jnp.dot(a_ref[...], yaug_ref[...], preferred_element_type=jnp.float32)
    z = zaug[:, :H]
    deg = zaug[:, H:H + 1]
    h = h0_ref[...] + z / jnp.maximum(deg, 1.0)
    norm = jnp.sqrt(jnp.maximum(jnp.sum(h * h, axis=-1, keepdims=True), 1e-12))
    h = jnp.maximum(h / norm, 0.0)
    psum = jnp.sum(h, axis=0, keepdims=True)

    @pl.when(i == 0)
    def _first():
        pooled_ref[...] = psum

    @pl.when(i > 0)
    def _rest():
        pooled_ref[...] += psum

    @pl.when(i == NR - 1)
    def _final():
        out_ref[...] = (jnp.dot(pooled_ref[...], wd_ref[...],
                                preferred_element_type=jnp.float32)
                        + bd_ref[...])


@functools.partial(jax.jit, static_argnames=())
def kernel(x, a, kernel, bias, Wd, bd):
    x = x.astype(jnp.float32)
    a = a.astype(jnp.float32)
    bias2 = bias.reshape(1, H)
    bd2 = bd.reshape(1, N_LABELS)

    yaug, h0 = pl.pallas_call(
        _prep_kernel,
        out_shape=(
            jax.ShapeDtypeStruct((N, 32), jnp.float32),
            jax.ShapeDtypeStruct((N, H), jnp.float32),
        ),
        in_specs=[
            pl.BlockSpec((N, F), lambda: (0, 0)),
            pl.BlockSpec((2 * F, H), lambda: (0, 0)),
            pl.BlockSpec((1, H), lambda: (0, 0)),
        ],
        out_specs=(
            pl.BlockSpec((N, 32), lambda: (0, 0)),
            pl.BlockSpec((N, H), lambda: (0, 0)),
        ),
    )(x, kernel, bias2)

    out = pl.pallas_call(
        _main_kernel,
        grid=(NR,),
        out_shape=jax.ShapeDtypeStruct((1, N_LABELS), jnp.float32),
        in_specs=[
            pl.BlockSpec((BR, N), lambda i: (i, 0)),
            pl.BlockSpec((N, 32), lambda i: (0, 0)),
            pl.BlockSpec((BR, H), lambda i: (i, 0)),
            pl.BlockSpec((H, N_LABELS), lambda i: (0, 0)),
            pl.BlockSpec((1, N_LABELS), lambda i: (0, 0)),
        ],
        out_specs=pl.BlockSpec((1, N_LABELS), lambda i: (0, 0)),
        scratch_shapes=[
            pltpu.VMEM((1, H), jnp.float32),
        ],
        compiler_params=pltpu.CompilerParams(
            dimension_semantics=("arbitrary",),
        ),
    )(a, yaug, h0, Wd, bd2)

    return out.reshape(N_LABELS)


# speedup vs baseline: 1.8264x; 1.0138x over previous
import functools

import jax
import jax.numpy as jnp
from jax.experimental import pallas as pl
from jax.experimental.pallas import tpu as pltpu

N = 10000
F = 128
H = 16
N_LABELS = 10

BR = 400
NR = N // BR


def _prep_kernel(x_ref, w_ref, b_ref, yaug_ref, h0_ref):
    x = x_ref[...]
    k1 = w_ref[:F, :]
    k2 = w_ref[F:, :]
    y = jnp.dot(x, k2, preferred_element_type=jnp.float32)
    yaug_ref[:, :H] = y
    yaug_ref[:, H:H + 1] = jnp.ones((N, 1), dtype=jnp.float32)
    yaug_ref[:, H + 1:] = jnp.zeros((N, 32 - H - 1), dtype=jnp.float32)
    h0_ref[...] = jnp.dot(x, k1, preferred_element_type=jnp.float32) + b_ref[...]


def _main_kernel(a_ref, yaug_ref, h0_ref, wd_ref, bd_ref, out_ref,
                 pooled_ref):
    i = pl.program_id(0)

    zaug = jnp.dot(a_ref[...], yaug_ref[...], preferred_element_type=jnp.float32)
    z = zaug[:, :H]
    deg = zaug[:, H:H + 1]
    h = h0_ref[...] + z / jnp.maximum(deg, 1.0)
    norm = jnp.sqrt(jnp.maximum(jnp.sum(h * h, axis=-1, keepdims=True), 1e-12))
    h = jnp.maximum(h / norm, 0.0)
    psum = jnp.sum(h, axis=0, keepdims=True)

    @pl.when(i == 0)
    def _first():
        pooled_ref[...] = psum

    @pl.when(i > 0)
    def _rest():
        pooled_ref[...] += psum

    @pl.when(i == NR - 1)
    def _final():
        out_ref[...] = (jnp.dot(pooled_ref[...], wd_ref[...],
                                preferred_element_type=jnp.float32)
                        + bd_ref[...])


@functools.partial(jax.jit, static_argnames=())
def kernel(x, a, kernel, bias, Wd, bd):
    x = x.astype(jnp.float32)
    a = a.astype(jnp.float32)
    bias2 = bias.reshape(1, H)
    bd2 = bd.reshape(1, N_LABELS)

    yaug, h0 = pl.pallas_call(
        _prep_kernel,
        out_shape=(
            jax.ShapeDtypeStruct((N, 32), jnp.float32),
            jax.ShapeDtypeStruct((N, H), jnp.float32),
        ),
        in_specs=[
            pl.BlockSpec((N, F), lambda: (0, 0)),
            pl.BlockSpec((2 * F, H), lambda: (0, 0)),
            pl.BlockSpec((1, H), lambda: (0, 0)),
        ],
        out_specs=(
            pl.BlockSpec((N, 32), lambda: (0, 0)),
            pl.BlockSpec((N, H), lambda: (0, 0)),
        ),
    )(x, kernel, bias2)

    out = pl.pallas_call(
        _main_kernel,
        grid=(NR,),
        out_shape=jax.ShapeDtypeStruct((1, N_LABELS), jnp.float32),
        in_specs=[
            pl.BlockSpec((BR, N), lambda i: (i, 0)),
            pl.BlockSpec((N, 32), lambda i: (0, 0)),
            pl.BlockSpec((BR, H), lambda i: (i, 0)),
            pl.BlockSpec((H, N_LABELS), lambda i: (0, 0)),
            pl.BlockSpec((1, N_LABELS), lambda i: (0, 0)),
        ],
        out_specs=pl.BlockSpec((1, N_LABELS), lambda i: (0, 0)),
        scratch_shapes=[
            pltpu.VMEM((1, H), jnp.float32),
        ],
        compiler_params=pltpu.CompilerParams(
            dimension_semantics=("arbitrary",),
        ),
    )(a, yaug, h0, Wd, bd2)

    return out.reshape(N_LABELS)


# speedup vs baseline: 2.0030x; 1.0967x over previous
import jax
import jax.numpy as jnp
from jax.experimental import pallas as pl
from jax.experimental.pallas import tpu as pltpu

N = 10000
F = 128
H = 16
N_LABELS = 10

BR = 400
NR = N // BR


def _main_kernel(a_ref, x_ref, w_ref, b_ref, wd_ref, bd_ref, out_ref,
                 yaug_ref, h0_ref, pooled_ref):
    i = pl.program_id(0)

    @pl.when(i == 0)
    def _prep():
        x = x_ref[...]
        yaug_ref[:, :H] = jnp.dot(x, w_ref[F:, :],
                                  preferred_element_type=jnp.float32)
        yaug_ref[:, H:H + 1] = jnp.ones((N, 1), dtype=jnp.float32)
        yaug_ref[:, H + 1:] = jnp.zeros((N, 32 - H - 1), dtype=jnp.float32)
        h0_ref[...] = jnp.dot(x, w_ref[:F, :],
                              preferred_element_type=jnp.float32) + b_ref[...]

    zaug = jnp.dot(a_ref[...], yaug_ref[...],
                   preferred_element_type=jnp.float32)
    z = zaug[:, :H]
    deg = zaug[:, H:H + 1]
    h = h0_ref[pl.ds(i * BR, BR), :] + z / jnp.maximum(deg, 1.0)
    norm = jnp.sqrt(jnp.maximum(jnp.sum(h * h, axis=-1, keepdims=True), 1e-12))
    h = jnp.maximum(h / norm, 0.0)
    psum = jnp.sum(h, axis=0, keepdims=True)

    @pl.when(i == 0)
    def _first():
        pooled_ref[...] = psum

    @pl.when(i > 0)
    def _rest():
        pooled_ref[...] += psum

    @pl.when(i == NR - 1)
    def _final():
        out_ref[...] = (jnp.dot(pooled_ref[...], wd_ref[...],
                                preferred_element_type=jnp.float32)
                        + bd_ref[...])


def kernel(x, a, kernel, bias, Wd, bd):
    x = x.astype(jnp.float32)
    a = a.astype(jnp.float32)
    bias2 = bias.reshape(1, H)
    bd2 = bd.reshape(1, N_LABELS)

    out = pl.pallas_call(
        _main_kernel,
        grid=(NR,),
        out_shape=jax.ShapeDtypeStruct((1, N_LABELS), jnp.float32),
        in_specs=[
            pl.BlockSpec((BR, N), lambda i: (i, 0)),
            pl.BlockSpec((N, F), lambda i: (0, 0)),
            pl.BlockSpec((2 * F, H), lambda i: (0, 0)),
            pl.BlockSpec((1, H), lambda i: (0, 0)),
            pl.BlockSpec((H, N_LABELS), lambda i: (0, 0)),
            pl.BlockSpec((1, N_LABELS), lambda i: (0, 0)),
        ],
        out_specs=pl.BlockSpec((1, N_LABELS), lambda i: (0, 0)),
        scratch_shapes=[
            pltpu.VMEM((N, 32), jnp.float32),
            pltpu.VMEM((N, H), jnp.float32),
            pltpu.VMEM((1, H), jnp.float32),
        ],
        compiler_params=pltpu.CompilerParams(
            dimension_semantics=("arbitrary",),
        ),
    )(a, x, kernel, bias2, Wd, bd2)

    return out.reshape(N_LABELS)
